# baseline (speedup 1.0000x reference)
import jax
import jax.numpy as jnp
from jax.experimental import pallas as pl
from jax.experimental.pallas import tpu as pltpu

_BM = 400
_NT = 10
_BT = _BM // _NT


def _manual_body(x_ref, w_ref, b_ref, adj_hbm, out_hbm, sup, ring, obuf,
                 asem, osem, asem_t, osem_t):
    n = x_ref.shape[0]
    nb_main = n // _BM - 1
    tail_base = nb_main * _BM
    tslot = (nb_main - 2) % 2

    for s in range(2):
        pltpu.make_async_copy(
            adj_hbm.at[pl.ds(s * _BM, _BM), :], ring.at[s], asem.at[s]
        ).start()
    sup[...] = jnp.dot(x_ref[...], w_ref[...], preferred_element_type=jnp.float32)

    def main_step(i, carry):
        slot = jax.lax.rem(i, 2)
        pltpu.make_async_copy(
            adj_hbm.at[pl.ds(i * _BM, _BM), :], ring.at[slot], asem.at[slot]
        ).wait()

        @pl.when(i >= 2)
        def _():
            pltpu.make_async_copy(
                obuf.at[slot], out_hbm.at[pl.ds((i - 2) * _BM, _BM), :],
                osem.at[slot],
            ).wait()

        obuf[slot] = (
            jnp.dot(ring[slot], sup[...], preferred_element_type=jnp.float32)
            + b_ref[...]
        )
        pltpu.make_async_copy(
            obuf.at[slot], out_hbm.at[pl.ds(i * _BM, _BM), :], osem.at[slot]
        ).start()

        @pl.when(i + 2 < nb_main)
        def _():
            pltpu.make_async_copy(
                adj_hbm.at[pl.ds((i + 2) * _BM, _BM), :], ring.at[slot],
                asem.at[slot],
            ).start()

        @pl.when(i == nb_main - 2)
        def _():
            for t in range(_NT):
                pltpu.make_async_copy(
                    adj_hbm.at[pl.ds(tail_base + t * _BT, _BT), :],
                    ring.at[tslot, pl.ds(t * _BT, _BT), :],
                    asem_t.at[t],
                ).start()

        return carry

    jax.lax.fori_loop(0, nb_main, main_step, 0)

    pltpu.make_async_copy(
        obuf.at[tslot], out_hbm.at[pl.ds((nb_main - 2) * _BM, _BM), :],
        osem.at[tslot],
    ).wait()
    for t in range(_NT):
        pltpu.make_async_copy(
            adj_hbm.at[pl.ds(tail_base + t * _BT, _BT), :],
            ring.at[tslot, pl.ds(t * _BT, _BT), :],
            asem_t.at[t],
        ).wait()
        obuf[tslot, pl.ds(t * _BT, _BT), :] = (
            jnp.dot(
                ring[tslot, pl.ds(t * _BT, _BT), :], sup[...],
                preferred_element_type=jnp.float32,
            )
            + b_ref[...]
        )
        pltpu.make_async_copy(
            obuf.at[tslot, pl.ds(t * _BT, _BT), :],
            out_hbm.at[pl.ds(tail_base + t * _BT, _BT), :],
            osem_t.at[t],
        ).start()

    other = 1 - tslot
    pltpu.make_async_copy(
        obuf.at[other], out_hbm.at[pl.ds((nb_main - 1) * _BM, _BM), :],
        osem.at[other],
    ).wait()
    for t in range(_NT):
        pltpu.make_async_copy(
            obuf.at[tslot, pl.ds(t * _BT, _BT), :],
            out_hbm.at[pl.ds(tail_base + t * _BT, _BT), :],
            osem_t.at[t],
        ).wait()


def _simple_body(x_ref, w_ref, b_ref, adj_ref, out_ref, sup_ref):
    @pl.when(pl.program_id(0) == 0)
    def _():
        sup_ref[...] = jnp.dot(
            x_ref[...], w_ref[...], preferred_element_type=jnp.float32
        )

    out_ref[...] = (
        jnp.dot(adj_ref[...], sup_ref[...], preferred_element_type=jnp.float32)
        + b_ref[...]
    )


def _simple_kernel(input, adj, weight, bias, bias2d, n, in_f, out_f):
    bm = 400 if n % 400 == 0 else n
    return pl.pallas_call(
        _simple_body,
        grid=(n // bm,),
        in_specs=[
            pl.BlockSpec((n, in_f), lambda i: (0, 0)),
            pl.BlockSpec((in_f, out_f), lambda i: (0, 0)),
            pl.BlockSpec((1, out_f), lambda i: (0, 0)),
            pl.BlockSpec((bm, n), lambda i: (i, 0)),
        ],
        out_specs=pl.BlockSpec((bm, out_f), lambda i: (i, 0)),
        out_shape=jax.ShapeDtypeStruct((n, out_f), jnp.float32),
        scratch_shapes=[pltpu.VMEM((n, out_f), jnp.float32)],
    )(input, weight, bias2d, adj)


def kernel(input, adj, weight, bias):
    n, in_f = input.shape
    out_f = weight.shape[1]
    bias2d = bias.reshape(1, out_f)
    if n % _BM != 0 or n // _BM < 3:
        return _simple_kernel(input, adj, weight, bias, bias2d, n, in_f, out_f)
    return pl.pallas_call(
        _manual_body,
        grid=(1,),
        in_specs=[
            pl.BlockSpec((n, in_f), lambda i: (0, 0)),
            pl.BlockSpec((in_f, out_f), lambda i: (0, 0)),
            pl.BlockSpec((1, out_f), lambda i: (0, 0)),
            pl.BlockSpec(memory_space=pltpu.MemorySpace.HBM),
        ],
        out_specs=pl.BlockSpec(memory_space=pltpu.MemorySpace.HBM),
        out_shape=jax.ShapeDtypeStruct((n, out_f), jnp.float32),
        scratch_shapes=[
            pltpu.VMEM((n, out_f), jnp.float32),
            pltpu.VMEM((2, _BM, n), jnp.float32),
            pltpu.VMEM((2, _BM, out_f), jnp.float32),
            pltpu.SemaphoreType.DMA((2,)),
            pltpu.SemaphoreType.DMA((2,)),
            pltpu.SemaphoreType.DMA((_NT,)),
            pltpu.SemaphoreType.DMA((_NT,)),
        ],
    )(input, weight, bias2d, adj)


# speedup vs baseline: 1.0708x; 1.0708x over previous
import jax
import jax.numpy as jnp
from jax.experimental import pallas as pl
from jax.experimental.pallas import tpu as pltpu

_BM = 400
_BH = _BM // 2


def _gcn_body(x_ref, w_ref, b_ref, adj_a, adj_b, out_ref, sup_ref):
    @pl.when(pl.program_id(0) == 0)
    def _():
        sup_ref[...] = jnp.dot(
            x_ref[...], w_ref[...], preferred_element_type=jnp.float32
        )

    out_ref[pl.ds(0, _BH), :] = (
        jnp.dot(adj_a[...], sup_ref[...], preferred_element_type=jnp.float32)
        + b_ref[...]
    )
    out_ref[pl.ds(_BH, _BH), :] = (
        jnp.dot(adj_b[...], sup_ref[...], preferred_element_type=jnp.float32)
        + b_ref[...]
    )


def kernel(input, adj, weight, bias):
    n, in_f = input.shape
    out_f = weight.shape[1]
    bm = _BM if n % _BM == 0 else n
    bh = bm // 2
    bias2d = bias.reshape(1, out_f)
    return pl.pallas_call(
        _gcn_body,
        grid=(n // bm,),
        in_specs=[
            pl.BlockSpec((n, in_f), lambda i: (0, 0)),
            pl.BlockSpec((in_f, out_f), lambda i: (0, 0)),
            pl.BlockSpec((1, out_f), lambda i: (0, 0)),
            pl.BlockSpec((bh, n), lambda i: (2 * i, 0)),
            pl.BlockSpec((bh, n), lambda i: (2 * i + 1, 0)),
        ],
        out_specs=pl.BlockSpec((bm, out_f), lambda i: (i, 0)),
        out_shape=jax.ShapeDtypeStruct((n, out_f), jnp.float32),
        scratch_shapes=[pltpu.VMEM((n, out_f), jnp.float32)],
    )(input, weight, bias2d, adj, adj)


# speedup vs baseline: 1.0941x; 1.0218x over previous
import jax
import jax.numpy as jnp
from jax.experimental import pallas as pl
from jax.experimental.pallas import tpu as pltpu

_BM = 400


def _gcn_body(x_ref, w_ref, b_ref, adj_ref, out_ref, sup_ref):
    @pl.when(pl.program_id(0) == 0)
    def _():
        sup_ref[...] = jnp.dot(
            x_ref[...], w_ref[...], preferred_element_type=jnp.float32
        )

    out_ref[...] = (
        jnp.dot(adj_ref[...], sup_ref[...], preferred_element_type=jnp.float32)
        + b_ref[...]
    )


def kernel(input, adj, weight, bias):
    n, in_f = input.shape
    out_f = weight.shape[1]
    bm = _BM if n % _BM == 0 else n
    bias2d = bias.reshape(1, out_f)
    return pl.pallas_call(
        _gcn_body,
        grid=(n // bm,),
        in_specs=[
            pl.BlockSpec((n, in_f), lambda i: (0, 0)),
            pl.BlockSpec((in_f, out_f), lambda i: (0, 0)),
            pl.BlockSpec((1, out_f), lambda i: (0, 0)),
            pl.BlockSpec((bm, n), lambda i: (i, 0)),
        ],
        out_specs=pl.BlockSpec((bm, out_f), lambda i: (i, 0)),
        out_shape=jax.ShapeDtypeStruct((n, out_f), jnp.float32),
        scratch_shapes=[pltpu.VMEM((n, out_f), jnp.float32)],
    )(input, weight, bias2d, adj)
